# initial kernel scaffold (unmeasured)
import jax
import jax.numpy as jnp
from jax import lax
from jax.experimental import pallas as pl
from jax.experimental.pallas import tpu as pltpu


def kernel(
    x,
):
    def body(*refs):
        pass

    out_shape = jax.ShapeDtypeStruct(..., jnp.float32)
    return pl.pallas_call(body, out_shape=out_shape)(...)


# baseline (device time: 122917 ns/iter reference)
import jax
import jax.numpy as jnp
from jax import lax
from jax.experimental import pallas as pl
from jax.experimental.pallas import tpu as pltpu

N_DEV = 16


def kernel(x):
    _, m, n_total = x.shape
    n_chunk = n_total // N_DEV

    def body(x_ref, out_ref, comm_ref, send_sems, recv_sems):
        my = lax.axis_index("i")
        left = lax.rem(my - 1 + N_DEV, N_DEV)
        right = lax.rem(my + 1, N_DEV)

        barrier_sem = pltpu.get_barrier_semaphore()
        for nbr in (left, right):
            pl.semaphore_signal(
                barrier_sem, inc=1,
                device_id=(nbr,), device_id_type=pl.DeviceIdType.MESH,
            )
        pl.semaphore_wait(barrier_sem, 2)

        def local_chunk(c):
            return x_ref[0, :, pl.ds(c * n_chunk, n_chunk)].astype(jnp.bfloat16)

        c0 = lax.rem(my - 1 + N_DEV, N_DEV)
        comm_ref[N_DEV - 1, :, :] = local_chunk(c0)

        for s in range(N_DEV - 1):
            src_slot = (N_DEV - 1) if s == 0 else (s - 1)
            if s > 0:
                c = lax.rem(my - 1 - s + 2 * N_DEV, N_DEV)
                comm_ref[s - 1, :, :] = comm_ref[s - 1, :, :] + local_chunk(c)
            rdma = pltpu.make_async_remote_copy(
                src_ref=comm_ref.at[src_slot],
                dst_ref=comm_ref.at[s],
                send_sem=send_sems.at[s],
                recv_sem=recv_sems.at[s],
                device_id=(right,),
                device_id_type=pl.DeviceIdType.MESH,
            )
            rdma.start()
            rdma.wait()

        out_ref[:, :] = (
            comm_ref[N_DEV - 2, :, :].astype(jnp.float32)
            + x_ref[0, :, pl.ds(my * n_chunk, n_chunk)].astype(jnp.float32)
        )

    return pl.pallas_call(
        body,
        out_shape=jax.ShapeDtypeStruct((m, n_chunk), jnp.float32),
        in_specs=[pl.BlockSpec(memory_space=pltpu.VMEM)],
        out_specs=pl.BlockSpec(memory_space=pltpu.VMEM),
        scratch_shapes=[
            pltpu.VMEM((N_DEV, m, n_chunk), jnp.bfloat16),
            pltpu.SemaphoreType.DMA((N_DEV - 1,)),
            pltpu.SemaphoreType.DMA((N_DEV - 1,)),
        ],
        compiler_params=pltpu.CompilerParams(collective_id=0),
    )(x)


# device time: 101124 ns/iter; 1.2155x vs baseline; 1.2155x over previous
import jax
import jax.numpy as jnp
from jax import lax
from jax.experimental import pallas as pl
from jax.experimental.pallas import tpu as pltpu

N_DEV = 16


def kernel(x):
    _, m, n_total = x.shape
    n_chunk = n_total // N_DEV
    m_half = m // 2

    def body(
        x_ref, out_ref,
        cw_ref, ccw_ref,
        cw_send, cw_recv, ccw_send, ccw_recv,
    ):
        my = lax.axis_index("i")
        left = lax.rem(my - 1 + N_DEV, N_DEV)
        right = lax.rem(my + 1, N_DEV)

        barrier_sem = pltpu.get_barrier_semaphore()
        for nbr in (left, right):
            pl.semaphore_signal(
                barrier_sem, inc=1,
                device_id=(nbr,), device_id_type=pl.DeviceIdType.MESH,
            )
        pl.semaphore_wait(barrier_sem, 2)

        def cw_chunk(c):
            return x_ref[0, 0:m_half, pl.ds(c * n_chunk, n_chunk)].astype(
                jnp.bfloat16
            )

        def ccw_chunk(c):
            return x_ref[0, m_half:m, pl.ds(c * n_chunk, n_chunk)].astype(
                jnp.bfloat16
            )

        cw_ref[N_DEV - 1, :, :] = cw_chunk(lax.rem(my - 1 + N_DEV, N_DEV))
        ccw_ref[N_DEV - 1, :, :] = ccw_chunk(lax.rem(my + 1, N_DEV))

        for s in range(N_DEV - 1):
            src_slot = (N_DEV - 1) if s == 0 else (s - 1)
            if s > 0:
                c_cw = lax.rem(my - 1 - s + 2 * N_DEV, N_DEV)
                c_ccw = lax.rem(my + 1 + s, N_DEV)
                cw_ref[s - 1, :, :] = cw_ref[s - 1, :, :] + cw_chunk(c_cw)
                ccw_ref[s - 1, :, :] = ccw_ref[s - 1, :, :] + ccw_chunk(c_ccw)
            rdma_cw = pltpu.make_async_remote_copy(
                src_ref=cw_ref.at[src_slot],
                dst_ref=cw_ref.at[s],
                send_sem=cw_send.at[s],
                recv_sem=cw_recv.at[s],
                device_id=(right,),
                device_id_type=pl.DeviceIdType.MESH,
            )
            rdma_ccw = pltpu.make_async_remote_copy(
                src_ref=ccw_ref.at[src_slot],
                dst_ref=ccw_ref.at[s],
                send_sem=ccw_send.at[s],
                recv_sem=ccw_recv.at[s],
                device_id=(left,),
                device_id_type=pl.DeviceIdType.MESH,
            )
            rdma_cw.start()
            rdma_ccw.start()
            rdma_cw.wait()
            rdma_ccw.wait()

        out_ref[0:m_half, :] = (
            cw_ref[N_DEV - 2, :, :].astype(jnp.float32)
            + x_ref[0, 0:m_half, pl.ds(my * n_chunk, n_chunk)]
        )
        out_ref[m_half:m, :] = (
            ccw_ref[N_DEV - 2, :, :].astype(jnp.float32)
            + x_ref[0, m_half:m, pl.ds(my * n_chunk, n_chunk)]
        )

    return pl.pallas_call(
        body,
        out_shape=jax.ShapeDtypeStruct((m, n_chunk), jnp.float32),
        in_specs=[pl.BlockSpec(memory_space=pltpu.VMEM)],
        out_specs=pl.BlockSpec(memory_space=pltpu.VMEM),
        scratch_shapes=[
            pltpu.VMEM((N_DEV, m_half, n_chunk), jnp.bfloat16),
            pltpu.VMEM((N_DEV, m_half, n_chunk), jnp.bfloat16),
            pltpu.SemaphoreType.DMA((N_DEV - 1,)),
            pltpu.SemaphoreType.DMA((N_DEV - 1,)),
            pltpu.SemaphoreType.DMA((N_DEV - 1,)),
            pltpu.SemaphoreType.DMA((N_DEV - 1,)),
        ],
        compiler_params=pltpu.CompilerParams(collective_id=0),
    )(x)


# device time: 79888 ns/iter; 1.5386x vs baseline; 1.2658x over previous
import jax
import jax.numpy as jnp
from jax import lax
from jax.experimental import pallas as pl
from jax.experimental.pallas import tpu as pltpu

N_DEV = 16


def kernel(x):
    _, m, n_total = x.shape
    n_chunk = n_total // N_DEV
    m_half = m // 2

    def body(
        x_ref, out_ref,
        cw_ref, ccw_ref,
        cw_send, cw_recv, ccw_send, ccw_recv,
    ):
        my = lax.axis_index("i")

        q = lax.rem(my, 4)
        z = my // 4
        r = jnp.where(
            q == 0, z,
            jnp.where(q == 1, 7 - z, jnp.where(q == 2, 8 + z, 15 - z)),
        )

        def dev_at(rho):
            rho = lax.rem(rho + 2 * N_DEV, N_DEV)
            col = rho // 4
            off = lax.rem(rho, 4)
            return jnp.where(
                col == 0, 4 * off,
                jnp.where(
                    col == 1, 4 * (3 - off) + 1,
                    jnp.where(col == 2, 4 * off + 2, 4 * (3 - off) + 3),
                ),
            )

        left = dev_at(r - 1)
        right = dev_at(r + 1)

        barrier_sem = pltpu.get_barrier_semaphore()
        for nbr in (left, right):
            pl.semaphore_signal(
                barrier_sem, inc=1,
                device_id=(nbr,), device_id_type=pl.DeviceIdType.MESH,
            )
        pl.semaphore_wait(barrier_sem, 2)

        def cw_chunk(c):
            return x_ref[0, 0:m_half, pl.ds(c * n_chunk, n_chunk)].astype(
                jnp.bfloat16
            )

        def ccw_chunk(c):
            return x_ref[0, m_half:m, pl.ds(c * n_chunk, n_chunk)].astype(
                jnp.bfloat16
            )

        cw_ref[N_DEV - 1, :, :] = cw_chunk(dev_at(r - 1))
        ccw_ref[N_DEV - 1, :, :] = ccw_chunk(dev_at(r + 1))

        for s in range(N_DEV - 1):
            src_slot = (N_DEV - 1) if s == 0 else (s - 1)
            if s > 0:
                c_cw = dev_at(r - 1 - s)
                c_ccw = dev_at(r + 1 + s)
                cw_ref[s - 1, :, :] = cw_ref[s - 1, :, :] + cw_chunk(c_cw)
                ccw_ref[s - 1, :, :] = ccw_ref[s - 1, :, :] + ccw_chunk(c_ccw)
            rdma_cw = pltpu.make_async_remote_copy(
                src_ref=cw_ref.at[src_slot],
                dst_ref=cw_ref.at[s],
                send_sem=cw_send.at[s],
                recv_sem=cw_recv.at[s],
                device_id=(right,),
                device_id_type=pl.DeviceIdType.MESH,
            )
            rdma_ccw = pltpu.make_async_remote_copy(
                src_ref=ccw_ref.at[src_slot],
                dst_ref=ccw_ref.at[s],
                send_sem=ccw_send.at[s],
                recv_sem=ccw_recv.at[s],
                device_id=(left,),
                device_id_type=pl.DeviceIdType.MESH,
            )
            rdma_cw.start()
            rdma_ccw.start()
            rdma_cw.wait()
            rdma_ccw.wait()

        out_ref[0:m_half, :] = (
            cw_ref[N_DEV - 2, :, :].astype(jnp.float32)
            + x_ref[0, 0:m_half, pl.ds(my * n_chunk, n_chunk)]
        )
        out_ref[m_half:m, :] = (
            ccw_ref[N_DEV - 2, :, :].astype(jnp.float32)
            + x_ref[0, m_half:m, pl.ds(my * n_chunk, n_chunk)]
        )

    return pl.pallas_call(
        body,
        out_shape=jax.ShapeDtypeStruct((m, n_chunk), jnp.float32),
        in_specs=[pl.BlockSpec(memory_space=pltpu.VMEM)],
        out_specs=pl.BlockSpec(memory_space=pltpu.VMEM),
        scratch_shapes=[
            pltpu.VMEM((N_DEV, m_half, n_chunk), jnp.bfloat16),
            pltpu.VMEM((N_DEV, m_half, n_chunk), jnp.bfloat16),
            pltpu.SemaphoreType.DMA((N_DEV - 1,)),
            pltpu.SemaphoreType.DMA((N_DEV - 1,)),
            pltpu.SemaphoreType.DMA((N_DEV - 1,)),
            pltpu.SemaphoreType.DMA((N_DEV - 1,)),
        ],
        compiler_params=pltpu.CompilerParams(collective_id=0),
    )(x)


# device time: 56283 ns/iter; 2.1839x vs baseline; 1.4194x over previous
import jax
import jax.numpy as jnp
from jax import lax
from jax.experimental import pallas as pl
from jax.experimental.pallas import tpu as pltpu

N_DEV = 16
K_SUB = 4


def kernel(x):
    _, m, n_total = x.shape
    n_chunk = n_total // N_DEV
    m_half = m // 2
    m_sub = m_half // K_SUB

    def body(
        x_ref, out_ref,
        cw_ref, ccw_ref,
        cw_send, cw_recv, ccw_send, ccw_recv,
    ):
        my = lax.axis_index("i")

        q = lax.rem(my, 4)
        z = my // 4
        r = jnp.where(
            q == 0, z,
            jnp.where(q == 1, 7 - z, jnp.where(q == 2, 8 + z, 15 - z)),
        )

        def dev_at(rho):
            rho = lax.rem(rho + 2 * N_DEV, N_DEV)
            col = rho // 4
            off = lax.rem(rho, 4)
            return jnp.where(
                col == 0, 4 * off,
                jnp.where(
                    col == 1, 4 * (3 - off) + 1,
                    jnp.where(col == 2, 4 * off + 2, 4 * (3 - off) + 3),
                ),
            )

        left = dev_at(r - 1)
        right = dev_at(r + 1)

        barrier_sem = pltpu.get_barrier_semaphore()
        for nbr in (left, right):
            pl.semaphore_signal(
                barrier_sem, inc=1,
                device_id=(nbr,), device_id_type=pl.DeviceIdType.MESH,
            )
        pl.semaphore_wait(barrier_sem, 2)

        def cw_x(c, j):
            return x_ref[
                0, j * m_sub:(j + 1) * m_sub, pl.ds(c * n_chunk, n_chunk)
            ].astype(jnp.bfloat16)

        def ccw_x(c, j):
            return x_ref[
                0,
                m_half + j * m_sub:m_half + (j + 1) * m_sub,
                pl.ds(c * n_chunk, n_chunk),
            ].astype(jnp.bfloat16)

        def rows(buf_slot, j):
            return buf_slot.at[pl.ds(j * m_sub, m_sub), :]

        def make(dirn, s, j):
            buf, send, recv, tgt = {
                "cw": (cw_ref, cw_send, cw_recv, right),
                "ccw": (ccw_ref, ccw_send, ccw_recv, left),
            }[dirn]
            src_slot = (N_DEV - 1) if s == 0 else (s - 1)
            return pltpu.make_async_remote_copy(
                src_ref=rows(buf.at[src_slot], j),
                dst_ref=rows(buf.at[s], j),
                send_sem=send.at[s, j],
                recv_sem=recv.at[s, j],
                device_id=(tgt,),
                device_id_type=pl.DeviceIdType.MESH,
            )

        cw_ref[N_DEV - 1, :, :] = (
            x_ref[0, 0:m_half, pl.ds(dev_at(r - 1) * n_chunk, n_chunk)]
        ).astype(jnp.bfloat16)
        ccw_ref[N_DEV - 1, :, :] = (
            x_ref[0, m_half:m, pl.ds(dev_at(r + 1) * n_chunk, n_chunk)]
        ).astype(jnp.bfloat16)

        rdmas = {}
        for j in range(K_SUB):
            for dirn in ("cw", "ccw"):
                rdmas[(dirn, 0, j)] = make(dirn, 0, j)
                rdmas[(dirn, 0, j)].start()

        for s in range(1, N_DEV - 1):
            c_cw = dev_at(r - 1 - s)
            c_ccw = dev_at(r + 1 + s)
            for j in range(K_SUB):
                lo = j * m_sub
                hi = (j + 1) * m_sub
                rdmas[("cw", s - 1, j)].wait_recv()
                cw_ref[s - 1, lo:hi, :] = (
                    cw_ref[s - 1, lo:hi, :] + cw_x(c_cw, j)
                )
                rdmas[("cw", s, j)] = make("cw", s, j)
                rdmas[("cw", s, j)].start()
                rdmas[("ccw", s - 1, j)].wait_recv()
                ccw_ref[s - 1, lo:hi, :] = (
                    ccw_ref[s - 1, lo:hi, :] + ccw_x(c_ccw, j)
                )
                rdmas[("ccw", s, j)] = make("ccw", s, j)
                rdmas[("ccw", s, j)].start()

        for j in range(K_SUB):
            lo = j * m_sub
            hi = (j + 1) * m_sub
            rdmas[("cw", N_DEV - 2, j)].wait_recv()
            out_ref[lo:hi, :] = (
                cw_ref[N_DEV - 2, lo:hi, :].astype(jnp.float32)
                + x_ref[0, lo:hi, pl.ds(my * n_chunk, n_chunk)]
            )
            rdmas[("ccw", N_DEV - 2, j)].wait_recv()
            out_ref[m_half + lo:m_half + hi, :] = (
                ccw_ref[N_DEV - 2, lo:hi, :].astype(jnp.float32)
                + x_ref[0, m_half + lo:m_half + hi, pl.ds(my * n_chunk, n_chunk)]
            )

        for key in rdmas:
            rdmas[key].wait_send()

    return pl.pallas_call(
        body,
        out_shape=jax.ShapeDtypeStruct((m, n_chunk), jnp.float32),
        in_specs=[pl.BlockSpec(memory_space=pltpu.VMEM)],
        out_specs=pl.BlockSpec(memory_space=pltpu.VMEM),
        scratch_shapes=[
            pltpu.VMEM((N_DEV, m_half, n_chunk), jnp.bfloat16),
            pltpu.VMEM((N_DEV, m_half, n_chunk), jnp.bfloat16),
            pltpu.SemaphoreType.DMA((N_DEV - 1, K_SUB)),
            pltpu.SemaphoreType.DMA((N_DEV - 1, K_SUB)),
            pltpu.SemaphoreType.DMA((N_DEV - 1, K_SUB)),
            pltpu.SemaphoreType.DMA((N_DEV - 1, K_SUB)),
        ],
        compiler_params=pltpu.CompilerParams(collective_id=0),
    )(x)
